# baseline (speedup 1.0000x reference)
import functools

import jax
import jax.numpy as jnp
from jax import lax
from jax.experimental import pallas as pl
from jax.experimental.pallas import tpu as pltpu
from jax.experimental.pallas import tpu_sc as plsc

SEQ = 200
BATCH = 4096
EMB = 64
VOCAB = 1000000
NC = 2
NS = 16
NW = NC * NS

BW = 256
NFULL = (VOCAB // BW)
VTAIL = NFULL * BW
NT = -(-NFULL // NW)

K = 128
NCHUNK = SEQ
NBUF = 4
LOOKAHEAD = 2


def _prep_body(tT_hbm, tail_hbm, tpad_hbm, vins, vouts, tbuf, rsem, wsem):
    wid = lax.axis_index("s") * NC + lax.axis_index("c")
    iota = lax.iota(jnp.int32, 16)
    zeros = jnp.zeros((16,), jnp.int32)

    def start_read(t, b):
        v0 = (wid + NW * t) * BW
        pltpu.async_copy(tT_hbm.at[:, pl.ds(v0, BW)], vins.at[b], rsem.at[b])

    def wait_read(t, b):
        v0 = (wid + NW * t) * BW
        pltpu.make_async_copy(tT_hbm.at[:, pl.ds(v0, BW)], vins.at[b],
                              rsem.at[b]).wait()

    def start_write(t, b):
        v0 = (wid + NW * t) * BW
        pltpu.async_copy(vouts.at[b], tpad_hbm.at[pl.ds(v0, BW)], wsem.at[b])

    def wait_write(t, b):
        v0 = (wid + NW * t) * BW
        pltpu.make_async_copy(vouts.at[b], tpad_hbm.at[pl.ds(v0, BW)],
                              wsem.at[b]).wait()

    def valid(t):
        return (wid + NW * t) < NFULL

    def transpose(b):
        @plsc.parallel_loop(0, BW // 16, unroll=2)
        def _(vb):
            ivl = iota + vb * 16
            for eb in range(4):
                erow = iota + eb * 16
                for d in range(16):
                    vls = (ivl + d) & (BW - 1)
                    vals = plsc.load_gather(vins.at[b], [erow, vls])
                    plsc.store_scatter(vouts.at[b], [vls, erow], vals)

    @pl.when(valid(0))
    def _():
        start_read(0, 0)

    @pl.when(valid(1))
    def _():
        start_read(1, 1)

    def outer(t2, _):
        for b in range(2):
            t = t2 * 2 + b

            @pl.when(valid(t))
            def _():
                wait_read(t, b)

                @pl.when(t >= 2)
                def _():
                    wait_write(t - 2, b)

                transpose(b)
                start_write(t, b)

                @pl.when(valid(t + 2))
                def _():
                    start_read(t + 2, b)
        return 0

    lax.fori_loop(0, (NT + 1) // 2, outer, 0)

    for tb in (NT - 3, NT - 2, NT - 1):
        @pl.when(valid(tb) & jnp.logical_not(valid(tb + 2)))
        def _():
            wait_write(tb, tb % 2)

    @pl.when(wid == 0)
    def _():
        pltpu.sync_copy(tail_hbm, tbuf)
        pltpu.sync_copy(tbuf, tpad_hbm.at[pl.ds(VTAIL, VOCAB - VTAIL)])


def _gather_body(x_hbm, tpad_hbm, out_hbm, idx_v, bufs, gsem, osem):
    wid = lax.axis_index("s") * NC + lax.axis_index("c")
    b0 = wid * K
    pltpu.sync_copy(x_hbm.at[:, pl.ds(b0, K)], idx_v)

    def start_gather(j, b):
        pltpu.async_copy(tpad_hbm.at[idx_v.at[j]], bufs.at[b], gsem.at[b])

    def wait_gather(j, b):
        pltpu.make_async_copy(tpad_hbm.at[idx_v.at[j]], bufs.at[b],
                              gsem.at[b]).wait()

    def start_write(j, b):
        pltpu.async_copy(bufs.at[b], out_hbm.at[pl.ds(j * BATCH + b0, K)],
                         osem.at[b])

    def wait_write(j, b):
        pltpu.make_async_copy(bufs.at[b], out_hbm.at[pl.ds(j * BATCH + b0, K)],
                              osem.at[b]).wait()

    for b in range(LOOKAHEAD):
        start_gather(b, b)

    def outer(g, _):
        for bi in range(NBUF):
            j = g * NBUF + bi
            bn = (bi + LOOKAHEAD) % NBUF
            jp = j + LOOKAHEAD - NBUF

            @pl.when(jp >= 0)
            def _():
                wait_write(jp, bn)

            @pl.when(j + LOOKAHEAD < NCHUNK)
            def _():
                start_gather(j + LOOKAHEAD, bn)

            wait_gather(j, bi)
            start_write(j, bi)
        return 0

    lax.fori_loop(0, NCHUNK // NBUF, outer, 0)

    for t in range(LOOKAHEAD):
        j = NCHUNK - LOOKAHEAD + t
        wait_write(j, j % NBUF)


_SC_MESH = dict(core_axis_name="c", subcore_axis_name="s")
_SC_PARAMS = dict(use_tc_tiling_on_sc=True, needs_layout_passes=False)


@jax.jit
def kernel(x, table):
    x32 = x.astype(jnp.int32)
    tT = table.T
    tail = jnp.pad(lax.slice(table, (VTAIL, 0), (VOCAB, EMB)),
                   ((0, 0), (0, 128 - EMB)))
    tpad = pl.kernel(
        _prep_body,
        out_type=jax.ShapeDtypeStruct((VOCAB, 128), jnp.float32),
        mesh=plsc.VectorSubcoreMesh(**_SC_MESH),
        scratch_types=[
            pltpu.VMEM((2, EMB, BW), jnp.float32),
            pltpu.VMEM((2, BW, 128), jnp.float32),
            pltpu.VMEM((VOCAB - VTAIL, 128), jnp.float32),
            pltpu.SemaphoreType.DMA((2,)),
            pltpu.SemaphoreType.DMA((2,)),
        ],
        compiler_params=pltpu.CompilerParams(**_SC_PARAMS),
    )(tT, tail)
    out = pl.kernel(
        _gather_body,
        out_type=jax.ShapeDtypeStruct((SEQ * BATCH, 128), jnp.float32),
        mesh=plsc.VectorSubcoreMesh(**_SC_MESH),
        scratch_types=[
            pltpu.VMEM((NCHUNK, K), jnp.int32),
            pltpu.VMEM((NBUF, K, 128), jnp.float32),
            pltpu.SemaphoreType.DMA((NBUF,)),
            pltpu.SemaphoreType.DMA((NBUF,)),
        ],
        compiler_params=pltpu.CompilerParams(**_SC_PARAMS),
    )(x32, tpad)
    return out[:, :EMB].reshape(SEQ, BATCH, EMB)


# speedup vs baseline: 1.3139x; 1.3139x over previous
import functools

import jax
import jax.numpy as jnp
from jax import lax
from jax.experimental import pallas as pl
from jax.experimental.pallas import tpu as pltpu
from jax.experimental.pallas import tpu_sc as plsc

SEQ = 200
BATCH = 4096
EMB = 64
VOCAB = 1000000
NC = 2
NS = 16
NW = NC * NS

K = 128
NCHUNK = SEQ
NBUF = 4
LOOKAHEAD = 2


def _gather_body(x_hbm, tpad_hbm, out_hbm, idx_v, bufs, gsem, osem):
    wid = lax.axis_index("s") * NC + lax.axis_index("c")
    b0 = wid * K
    pltpu.sync_copy(x_hbm.at[:, pl.ds(b0, K)], idx_v)

    def start_gather(j, b):
        pltpu.async_copy(tpad_hbm.at[idx_v.at[j]], bufs.at[b], gsem.at[b])

    def wait_gather(j, b):
        pltpu.make_async_copy(tpad_hbm.at[idx_v.at[j]], bufs.at[b],
                              gsem.at[b]).wait()

    def start_write(j, b):
        pltpu.async_copy(bufs.at[b], out_hbm.at[pl.ds(j * BATCH + b0, K)],
                         osem.at[b])

    def wait_write(j, b):
        pltpu.make_async_copy(bufs.at[b], out_hbm.at[pl.ds(j * BATCH + b0, K)],
                              osem.at[b]).wait()

    for b in range(LOOKAHEAD):
        start_gather(b, b)

    def outer(g, _):
        for bi in range(NBUF):
            j = g * NBUF + bi
            bn = (bi + LOOKAHEAD) % NBUF
            jp = j + LOOKAHEAD - NBUF

            @pl.when(jp >= 0)
            def _():
                wait_write(jp, bn)

            @pl.when(j + LOOKAHEAD < NCHUNK)
            def _():
                start_gather(j + LOOKAHEAD, bn)

            wait_gather(j, bi)
            start_write(j, bi)
        return 0

    lax.fori_loop(0, NCHUNK // NBUF, outer, 0)

    for t in range(LOOKAHEAD):
        j = NCHUNK - LOOKAHEAD + t
        wait_write(j, j % NBUF)


_SC_MESH = dict(core_axis_name="c", subcore_axis_name="s")
_SC_PARAMS = dict(use_tc_tiling_on_sc=True, needs_layout_passes=False)


@jax.jit
def kernel(x, table):
    x32 = x.astype(jnp.int32)
    tpad = jnp.pad(table, ((0, 0), (0, 128 - EMB)))
    out = pl.kernel(
        _gather_body,
        out_type=jax.ShapeDtypeStruct((SEQ * BATCH, 128), jnp.float32),
        mesh=plsc.VectorSubcoreMesh(**_SC_MESH),
        scratch_types=[
            pltpu.VMEM((NCHUNK, K), jnp.int32),
            pltpu.VMEM((NBUF, K, 128), jnp.float32),
            pltpu.SemaphoreType.DMA((NBUF,)),
            pltpu.SemaphoreType.DMA((NBUF,)),
        ],
        compiler_params=pltpu.CompilerParams(**_SC_PARAMS),
    )(x32, tpad)
    return out[:, :EMB].reshape(SEQ, BATCH, EMB)


# speedup vs baseline: 1.3156x; 1.0013x over previous
import jax
import jax.numpy as jnp
from jax import lax
from jax.experimental import pallas as pl
from jax.experimental.pallas import tpu as pltpu
from jax.experimental.pallas import tpu_sc as plsc

SEQ = 200
BATCH = 4096
EMB = 64
VOCAB = 1000000
NC = 2
NS = 16
NW = NC * NS

K = 128
NCHUNK = SEQ
NBUF = 4
LOOKAHEAD = 2


def _gather_body(x_hbm, tpad_hbm, out_hbm, idx_v, bufs, gsem, osem):
    wid = lax.axis_index("s") * NC + lax.axis_index("c")
    b0 = wid * K
    pltpu.sync_copy(x_hbm.at[:, pl.ds(b0, K)], idx_v)

    def start_gather(j, b):
        pltpu.async_copy(tpad_hbm.at[idx_v.at[j]], bufs.at[b], gsem.at[b])

    def wait_gather(j, b):
        pltpu.make_async_copy(tpad_hbm.at[idx_v.at[j]], bufs.at[b],
                              gsem.at[b]).wait()

    def start_write(j, b):
        pltpu.async_copy(bufs.at[b], out_hbm.at[pl.ds(j * BATCH + b0, K)],
                         osem.at[b])

    def wait_write(j, b):
        pltpu.make_async_copy(bufs.at[b], out_hbm.at[pl.ds(j * BATCH + b0, K)],
                              osem.at[b]).wait()

    for b in range(LOOKAHEAD):
        start_gather(b, b)

    def outer(g, _):
        for bi in range(NBUF):
            j = g * NBUF + bi
            bn = (bi + LOOKAHEAD) % NBUF
            jp = j + LOOKAHEAD - NBUF

            @pl.when(jp >= 0)
            def _():
                wait_write(jp, bn)

            @pl.when(j + LOOKAHEAD < NCHUNK)
            def _():
                start_gather(j + LOOKAHEAD, bn)

            wait_gather(j, bi)
            start_write(j, bi)
        return 0

    lax.fori_loop(0, NCHUNK // NBUF, outer, 0)

    for t in range(LOOKAHEAD):
        j = NCHUNK - LOOKAHEAD + t
        wait_write(j, j % NBUF)


_SC_MESH = dict(core_axis_name="c", subcore_axis_name="s")
_SC_PARAMS = dict(use_tc_tiling_on_sc=True, needs_layout_passes=False)


@jax.jit
def kernel(x, table):
    x32 = x.astype(jnp.int32)
    tpad = jnp.pad(table, ((0, 0), (0, 128 - EMB)))
    out = pl.kernel(
        _gather_body,
        out_type=jax.ShapeDtypeStruct((SEQ * BATCH, 128), jnp.float32),
        mesh=plsc.VectorSubcoreMesh(**_SC_MESH),
        scratch_types=[
            pltpu.VMEM((NCHUNK, K), jnp.int32),
            pltpu.VMEM((NBUF, K, 128), jnp.float32),
            pltpu.SemaphoreType.DMA((NBUF,)),
            pltpu.SemaphoreType.DMA((NBUF,)),
        ],
        compiler_params=pltpu.CompilerParams(**_SC_PARAMS),
    )(x32, tpad)
    return out[:, :EMB].reshape(SEQ, BATCH, EMB)
